# Initial kernel scaffold; baseline (speedup 1.0000x reference)
import jax
import jax.numpy as jnp
from jax.experimental import pallas as pl


def kernel(x, edge_index, W1, b1, W2, b2):
    raise NotImplementedError("write your pallas kernel here")


# speedup vs baseline: 3.2233x; 3.2233x over previous
import functools

import jax
import jax.numpy as jnp
from jax import lax
from jax.experimental import pallas as pl
from jax.experimental.pallas import tpu as pltpu
from jax.experimental.pallas import tpu_sc as plsc

N = 10000
E = 320000
D = 128
NS = 16
CH = 48
G = 8
NK = 424
EP = NS * NK * CH
NGRP = NK // G
NP = 10240
PADNODE = NP - 1
HR = NP // 2
AR = 5248
RPT = NP // NS
HRT = HR // NS
ART = AR // NS


@functools.lru_cache(maxsize=None)
def _mesh():
    return plsc.VectorSubcoreMesh(
        core_axis_name="c", subcore_axis_name="s", num_cores=1)


def _fill_vmem(ref, nrows, ncols, value):
    vec = jnp.full((16,), value, jnp.float32)

    def row(i, carry):
        def col(l, inner):
            ref[i, pl.ds(l * 16, 16)] = vec
            return inner
        return lax.fori_loop(0, ncols // 16, col, carry)

    lax.fori_loop(0, nrows, row, 0)


def _zero_acc(acc, zbuf_v, s, width):
    del width
    for k in range(ART // 8):
        pltpu.sync_copy(zbuf_v, acc.at[pl.ds(s * ART + k * 8, 8)])


def _remap_group(dstv, dstm, r):
    for j in range(G):
        for l in range(CH // 16):
            v = dstv[j, pl.ds(l * 16, 16)]
            if r == 0:
                m = jnp.minimum(v, HR)
            else:
                m = jnp.maximum(v, HR - 8) - (HR - 8)
            dstm[j, pl.ds(l * 16, 16)] = m


@functools.lru_cache(maxsize=None)
def _deg_kernel():
    return pl.kernel(
        _deg_body,
        out_type=jax.ShapeDtypeStruct((NS * NP,), jnp.float32),
        mesh=_mesh(),
        compiler_params=pltpu.CompilerParams(needs_layout_passes=False),
        scratch_types=[
            pltpu.VMEM((G * CH // 16, 16), jnp.int32),
            pltpu.VMEM((NP,), jnp.float32),
        ],
    )


_GR = G * CH // 16


def _deg_body(dst_hbm, out_hbm, dstv, hist):
    s = lax.axis_index("s")
    zero = jnp.zeros((16,), jnp.float32)

    def zrow(i, c):
        hist[pl.ds(i * 16, 16)] = zero
        return c
    lax.fori_loop(0, NP // 16, zrow, 0)

    ones = jnp.ones((16,), jnp.float32)

    def group(g, carry):
        pltpu.sync_copy(dst_hbm.at[s, pl.ds(g * _GR, _GR)], dstv)
        for j in range(_GR):
            v = dstv[j, :]
            plsc.addupdate_scatter(hist, [v], ones)
        return carry

    lax.fori_loop(0, NGRP, group, 0)
    pltpu.sync_copy(hist, out_hbm.at[pl.ds(s * NP, NP)])


@functools.lru_cache(maxsize=None)
def _agg_kernel(npasses):
    return pl.kernel(
        _agg_body_factory(npasses),
        out_type=jax.ShapeDtypeStruct((npasses * NP, D), jnp.float32),
        mesh=_mesh(),
        scratch_types=[
            pltpu.VMEM((G, CH), jnp.int32),
            pltpu.VMEM((G, CH), jnp.int32),
            pltpu.VMEM((G, CH), jnp.int32),
            pltpu.VMEM((CH, D), jnp.float32),
            pltpu.VMEM((CH, D), jnp.float32),
            pltpu.VMEM((8, D), jnp.float32),
            pltpu.VMEM_SHARED((AR, D), jnp.float32),
            pltpu.SemaphoreType.DMA,
            pltpu.SemaphoreType.DMA,
        ],
    )


def _agg_body_factory(npasses):
    def body(*refs):
        tables = refs[:npasses]
        (src_hbm, dst_hbm, out_hbm, srcv, dstv, dstm,
         rows0, rows1, zbuf_v, acc, sem0, sem1) = refs[npasses:]
        s = lax.axis_index("s")
        bufs = (rows0, rows1)
        sems = (sem0, sem1)
        _fill_vmem(zbuf_v, 8, D, 0.0)

        for p in range(npasses):
            table = tables[p]
            for r in range(2):
                _zero_acc(acc, zbuf_v, s, D)
                plsc.subcore_barrier()

                def group(g, carry, r=r, table=table):
                    pltpu.sync_copy(src_hbm.at[s, pl.ds(g * G, G)], srcv)
                    pltpu.sync_copy(dst_hbm.at[s, pl.ds(g * G, G)], dstv)
                    _remap_group(dstv, dstm, r)
                    pltpu.async_copy(table.at[srcv.at[0]], bufs[0], sems[0])
                    for j in range(G):
                        b = j % 2
                        if j + 1 < G:
                            pltpu.async_copy(
                                table.at[srcv.at[j + 1]],
                                bufs[(j + 1) % 2], sems[(j + 1) % 2])
                        pltpu.make_async_copy(
                            table.at[srcv.at[j]], bufs[b], sems[b]).wait()
                        pltpu.sync_copy(bufs[b], acc.at[dstm.at[j]], add=True)
                    return carry

                lax.fori_loop(0, NGRP, group, 0)
                plsc.subcore_barrier()
                base = 0 if r == 0 else 8
                pltpu.sync_copy(
                    acc.at[pl.ds(base + s * HRT, HRT)],
                    out_hbm.at[pl.ds(p * NP + r * HR + s * HRT, HRT)])
                plsc.subcore_barrier()
    return body


def _dinv_from_counts(degc):
    deg = jnp.sum(degc.reshape(NS, NP), axis=0)
    return lax.rsqrt(deg[:N] + 1.0)


_PAD = NP - N


def _padded(h):
    return jnp.concatenate(
        [h, jnp.zeros((_PAD, h.shape[1]), jnp.float32)], axis=0)


def _tc1_body(x_ref, w1_ref, degc_ref, out_ref):
    dinv = _dinv_from_counts(degc_ref[...])
    h = jnp.dot(x_ref[...], w1_ref[...], preferred_element_type=jnp.float32)
    out_ref[...] = _padded(h * dinv[:, None])


def _tc2_body(tmp1_ref, g1_ref, degc_ref, b1_ref, w2_ref, t0_ref, t1_ref):
    dinv = _dinv_from_counts(degc_ref[...])
    out1 = jax.nn.relu(
        (tmp1_ref[:N, :] + g1_ref[:N, :]) * dinv[:, None] + b1_ref[...])
    h2 = jnp.dot(out1, w2_ref[...], preferred_element_type=jnp.float32)
    h2 = h2 * dinv[:, None]
    t0_ref[...] = _padded(h2[:, :D])
    t1_ref[...] = _padded(h2[:, D:])


def _tc3_body(tmp2_ref, t0_ref, t1_ref, degc_ref, b2_ref, out_ref):
    dinv = _dinv_from_counts(degc_ref[...])
    b2 = b2_ref[...]
    o0 = jax.nn.relu(
        (tmp2_ref[:N, :] + t0_ref[:N, :]) * dinv[:, None] + b2[:D])
    o1 = jax.nn.relu(
        (tmp2_ref[NP:NP + N, :] + t1_ref[:N, :]) * dinv[:, None] + b2[D:])
    out_ref[...] = jnp.concatenate(
        [jnp.mean(o0, axis=0), jnp.mean(o1, axis=0)])


_f32 = jnp.float32


def kernel(x, edge_index, W1, b1, W2, b2):
    src = jnp.concatenate(
        [edge_index[0], jnp.zeros((EP - E,), jnp.int32)])
    dst = jnp.concatenate(
        [edge_index[1], jnp.full((EP - E,), PADNODE, jnp.int32)])
    srce = src.reshape(NS, NK, CH)
    dste = dst.reshape(NS, NK, CH)
    dste16 = dst.reshape(NS, NK * CH // 16, 16)

    degc = _deg_kernel()(dste16)

    g1 = pl.pallas_call(
        _tc1_body,
        out_shape=jax.ShapeDtypeStruct((NP, D), _f32),
    )(x, W1, degc)

    tmp1 = _agg_kernel(1)(g1, srce, dste)

    t0, t1 = pl.pallas_call(
        _tc2_body,
        out_shape=(jax.ShapeDtypeStruct((NP, D), _f32),
                   jax.ShapeDtypeStruct((NP, D), _f32)),
    )(tmp1, g1, degc, b1, W2)

    tmp2 = _agg_kernel(2)(t0, t1, srce, dste)

    out = pl.pallas_call(
        _tc3_body,
        out_shape=jax.ShapeDtypeStruct((2 * D,), _f32),
    )(tmp2, t0, t1, degc, b2)
    return out
